# Initial kernel scaffold; baseline (speedup 1.0000x reference)
import jax
import jax.numpy as jnp
from jax.experimental import pallas as pl


def kernel(x, edge_index, W_lin, att_src, att_dst, bias_gat, W_ih, W_hh, b_ih, b_hh):
    raise NotImplementedError("write your pallas kernel here")


# speedup vs baseline: 8.7543x; 8.7543x over previous
import functools

import jax
import jax.numpy as jnp
from jax import lax
from jax.experimental import pallas as pl
from jax.experimental.pallas import tpu as pltpu
from jax.experimental.pallas import tpu_sc as plsc

N = 10000
E = 160000
D = 256
DH = 128
NP = 10240
BN = 2000
C = 80
G = 8
NCH = 128
EPT = NCH * C
EPAD = 16 * EPT
ROWS_T = 625


def _tc1_body(x_ref, wt_ref, at_ref, h0_ref, h1_ref, a_ref):
    h = jax.lax.dot_general(
        x_ref[...], wt_ref[...], (((1,), (0,)), ((), ())),
        preferred_element_type=jnp.float32)
    h0_ref[...] = h[:, :DH]
    h1_ref[...] = h[:, DH:]
    a_ref[...] = jax.lax.dot_general(
        h, at_ref[...], (((1,), (0,)), ((), ())),
        preferred_element_type=jnp.float32)


def _tc1(x, w_t, att_t):
    nblk = N // BN
    return pl.pallas_call(
        _tc1_body,
        grid=(nblk,),
        in_specs=[
            pl.BlockSpec((BN, D), lambda i: (i, 0)),
            pl.BlockSpec((D, D), lambda i: (0, 0)),
            pl.BlockSpec((D, DH), lambda i: (0, 0)),
        ],
        out_specs=[
            pl.BlockSpec((BN, DH), lambda i: (i, 0)),
            pl.BlockSpec((BN, DH), lambda i: (i, 0)),
            pl.BlockSpec((BN, DH), lambda i: (i, 0)),
        ],
        out_shape=[
            jax.ShapeDtypeStruct((N, DH), jnp.float32),
            jax.ShapeDtypeStruct((N, DH), jnp.float32),
            jax.ShapeDtypeStruct((N, DH), jnp.float32),
        ],
    )(x, w_t, att_t)


def _sc_agg(a_src, a_dst, src3, dst3, h0, h1):
    mesh = plsc.VectorSubcoreMesh(core_axis_name="c", subcore_axis_name="s")

    @functools.partial(
        pl.kernel,
        mesh=mesh,
        compiler_params=pltpu.CompilerParams(needs_layout_passes=False),
        out_type=[
            jax.ShapeDtypeStruct((16, ROWS_T, DH), jnp.float32),
            jax.ShapeDtypeStruct((16, ROWS_T, DH), jnp.float32),
        ],
        scratch_types=[
            pltpu.VMEM((NP,), jnp.float32),
            pltpu.VMEM((NP,), jnp.float32),
            pltpu.VMEM((NP // 128, 128), jnp.float32),
            pltpu.VMEM((C, DH), jnp.float32),
            pltpu.VMEM((G, C), jnp.int32),
            pltpu.VMEM((G, C), jnp.int32),
            pltpu.VMEM((1, 80), jnp.int32),
            pltpu.VMEM((5, 128), jnp.float32),
            pltpu.VMEM_SHARED((NP // 128, 128), jnp.float32),
            pltpu.VMEM_SHARED((N, DH), jnp.float32),
            pltpu.SemaphoreType.DMA,
        ],
    )
    def k(a_src_h, a_dst_h, src_h, dst_h, h0_h, h1_h, o0_h, o1_h,
          asrc_v, adst_v, den_v, rows_v, src_g, dst_g, iota_v, red_v,
          den_s, acc_s, sem):
        c = lax.axis_index("c")
        s = lax.axis_index("s")

        pltpu.sync_copy(a_src_h, asrc_v.at[pl.ds(0, N)])
        pltpu.sync_copy(a_dst_h, adst_v.at[pl.ds(0, N)])

        def _lanemax(v):
            m = v[0]
            for i in range(1, 16):
                m = jnp.maximum(m, v[i])
            return m

        def _maxloop(i, m):
            return jnp.maximum(m, asrc_v[pl.ds(i * 16, 16)])
        ms = _lanemax(lax.fori_loop(0, N // 16, _maxloop,
                                    jnp.full((16,), -3e38, jnp.float32)))

        def _maxloop2(i, m):
            return jnp.maximum(m, adst_v[pl.ds(i * 16, 16)])
        md = _lanemax(lax.fori_loop(0, N // 16, _maxloop2,
                                    jnp.full((16,), -3e38, jnp.float32)))
        t = ms + md
        g = jnp.where(t > 0, t, 0.2 * t)

        zero16 = jnp.zeros((16,), jnp.float32)

        def _zden(i, _):
            for kk in range(8):
                den_v[i, pl.ds(kk * 16, 16)] = zero16
            return 0
        lax.fori_loop(0, NP // 128, _zden, 0)

        def _zrow(i, _):
            for kk in range(DH // 16):
                rows_v[i, pl.ds(kk * 16, 16)] = zero16
            return 0
        lax.fori_loop(0, C, _zrow, 0)

        lanes = lax.iota(jnp.int32, 16)
        for i in range(5):
            iota_v[0, pl.ds(i * 16, 16)] = lanes + i * 16

        def _zred(i, _):
            for kk in range(8):
                red_v[i, pl.ds(kk * 16, 16)] = zero16
            return 0
        lax.fori_loop(0, 5, _zred, 0)

        @pl.when(s == 0)
        def _():
            for kk in range(16):
                pltpu.sync_copy(red_v, den_s.at[pl.ds(kk * 5, 5)])
        for m in range(7):
            pltpu.sync_copy(rows_v,
                            acc_s.at[pl.ds(s * ROWS_T + m * C, C)])
        pltpu.sync_copy(rows_v.at[pl.ds(0, 65)],
                        acc_s.at[pl.ds(s * ROWS_T + 560, 65)])

        ebase = s * EPT

        def _p1g(gi, _):
            pltpu.sync_copy(src_h.at[s, pl.ds(gi * G, G)], src_g)
            pltpu.sync_copy(dst_h.at[s, pl.ds(gi * G, G)], dst_g)

            def _p1c(j, _):
                cbase = ebase + (gi * G + j) * C
                for i in range(C // 16):
                    sl = pl.ds(i * 16, 16)
                    si = src_g[j, sl]
                    di = dst_g[j, sl]
                    msk = (cbase + i * 16 + lanes) < E
                    al = (plsc.load_gather(asrc_v, [si])
                          + plsc.load_gather(adst_v, [di]))
                    al = jnp.where(al > 0, al, 0.2 * al)
                    ex = jnp.exp(al - g)
                    plsc.addupdate_scatter(
                        den_v, [jnp.right_shift(di, 7),
                                jnp.bitwise_and(di, 127)], ex, mask=msk)
                return 0
            lax.fori_loop(0, G, _p1c, 0)
            return 0
        lax.fori_loop(0, NCH // G, _p1g, 0)

        plsc.subcore_barrier()

        pltpu.sync_copy(den_v, den_s.at[iota_v.at[0]], add=True)
        plsc.subcore_barrier()

        pltpu.sync_copy(den_s.at[pl.ds(s * 5, 5)], red_v)

        def _recip(i, _):
            for kk in range(8):
                sl = pl.ds(kk * 16, 16)
                red_v[i, sl] = 1.0 / (red_v[i, sl] + 1e-16)
            return 0
        lax.fori_loop(0, 5, _recip, 0)
        pltpu.sync_copy(red_v, den_s.at[pl.ds(s * 5, 5)])
        plsc.subcore_barrier()

        pltpu.sync_copy(den_s, den_v)

        def _p2(h_h):
            def _p2g(gi, _):
                pltpu.sync_copy(src_h.at[s, pl.ds(gi * G, G)], src_g)
                pltpu.sync_copy(dst_h.at[s, pl.ds(gi * G, G)], dst_g)

                def _p2c(j, _):
                    pltpu.async_copy(h_h.at[src_g.at[j]], rows_v, sem).wait()
                    cbase = ebase + (gi * G + j) * C

                    def _scale(i, _):
                        sl = pl.ds(i * 16, 16)
                        si = src_g[j, sl]
                        di = dst_g[j, sl]
                        msk = (cbase + i * 16 + lanes) < E
                        al = (plsc.load_gather(asrc_v, [si])
                              + plsc.load_gather(adst_v, [di]))
                        al = jnp.where(al > 0, al, 0.2 * al)
                        ex = jnp.exp(al - g)
                        rd = plsc.load_gather(
                            den_v, [jnp.right_shift(di, 7),
                                    jnp.bitwise_and(di, 127)])
                        att16 = jnp.where(msk, ex * rd, 0.0)
                        for e2 in range(16):
                            a = att16[e2]
                            e = i * 16 + e2
                            for kk in range(DH // 16):
                                slk = pl.ds(kk * 16, 16)
                                rows_v[e, slk] = rows_v[e, slk] * a
                        return 0
                    lax.fori_loop(0, C // 16, _scale, 0)
                    pltpu.sync_copy(rows_v, acc_s.at[dst_g.at[j]], add=True)
                    return 0
                lax.fori_loop(0, G, _p2c, 0)
                return 0
            lax.fori_loop(0, NCH // G, _p2g, 0)

        @pl.when(c == 0)
        def _():
            _p2(h0_h)

        @pl.when(c == 1)
        def _():
            _p2(h1_h)

        plsc.subcore_barrier()

        @pl.when(c == 0)
        def _():
            pltpu.sync_copy(acc_s.at[pl.ds(s * ROWS_T, ROWS_T)], o0_h.at[s])

        @pl.when(c == 1)
        def _():
            pltpu.sync_copy(acc_s.at[pl.ds(s * ROWS_T, ROWS_T)], o1_h.at[s])

    return k(a_src, a_dst, src3, dst3, h0, h1)


def _tc2_body(g0_ref, g1_ref, x_ref, bg_ref, wih_t_ref, whh_t_ref,
              bih_ref, bhh_ref, out_ref):
    e0 = g0_ref[...] + bg_ref[0, :DH][None, :]
    e1 = g1_ref[...] + bg_ref[0, DH:][None, :]
    h0 = jnp.where(e0 > 0, e0, jnp.exp(e0) - 1.0)
    h1 = jnp.where(e1 > 0, e1, jnp.exp(e1) - 1.0)
    wih_t = wih_t_ref[...]
    gi = (jax.lax.dot_general(h0, wih_t[:DH, :], (((1,), (0,)), ((), ())),
                              preferred_element_type=jnp.float32)
          + jax.lax.dot_general(h1, wih_t[DH:, :], (((1,), (0,)), ((), ())),
                                preferred_element_type=jnp.float32)
          + bih_ref[0, :][None, :])
    x = x_ref[...]
    gh = (jax.lax.dot_general(x, whh_t_ref[...], (((1,), (0,)), ((), ())),
                              preferred_element_type=jnp.float32)
          + bhh_ref[0, :][None, :])
    r = jax.nn.sigmoid(gi[:, :D] + gh[:, :D])
    z = jax.nn.sigmoid(gi[:, D:2 * D] + gh[:, D:2 * D])
    n = jnp.tanh(gi[:, 2 * D:] + r * gh[:, 2 * D:])
    out_ref[...] = (1.0 - z) * n + z * x


def _tc2(g0, g1, x, bias_gat, wih_t, whh_t, b_ih, b_hh):
    nblk = N // BN
    return pl.pallas_call(
        _tc2_body,
        grid=(nblk,),
        in_specs=[
            pl.BlockSpec((BN, DH), lambda i: (i, 0)),
            pl.BlockSpec((BN, DH), lambda i: (i, 0)),
            pl.BlockSpec((BN, D), lambda i: (i, 0)),
            pl.BlockSpec((1, D), lambda i: (0, 0)),
            pl.BlockSpec((D, 3 * D), lambda i: (0, 0)),
            pl.BlockSpec((D, 3 * D), lambda i: (0, 0)),
            pl.BlockSpec((1, 3 * D), lambda i: (0, 0)),
            pl.BlockSpec((1, 3 * D), lambda i: (0, 0)),
        ],
        out_specs=pl.BlockSpec((BN, D), lambda i: (i, 0)),
        out_shape=jax.ShapeDtypeStruct((N, D), jnp.float32),
    )(g0, g1, x, bias_gat, wih_t, whh_t, b_ih, b_hh)


def kernel(x, edge_index, W_lin, att_src, att_dst, bias_gat,
           W_ih, W_hh, b_ih, b_hh):
    w_t = W_lin.T
    att_t = jnp.zeros((D, DH), jnp.float32)
    att_t = att_t.at[:, 0].set(att_src).at[:, 1].set(att_dst)
    h0, h1, a_mat = _tc1(x, w_t, att_t)
    a_src = a_mat[:, 0]
    a_dst = a_mat[:, 1]
    pad = jnp.zeros((EPAD - E,), jnp.int32)
    src3 = jnp.concatenate([edge_index[0], pad]).reshape(16, NCH, C)
    dst3 = jnp.concatenate([edge_index[1], pad]).reshape(16, NCH, C)
    g0, g1 = _sc_agg(a_src, a_dst, src3, dst3, h0, h1)
    g0 = g0.reshape(N, DH)
    g1 = g1.reshape(N, DH)
    return _tc2(g0, g1, x, bias_gat.reshape(1, D), W_ih.T, W_hh.T,
                b_ih.reshape(1, 3 * D), b_hh.reshape(1, 3 * D))
